# initial kernel scaffold (unmeasured)
import jax
import jax.numpy as jnp
from jax import lax
from jax.experimental import pallas as pl
from jax.experimental.pallas import tpu as pltpu

N_DEV = 4


def kernel(x, w_mat):
    m, k = x.shape
    _, n = w_mat.shape
    m_chunk = m // N_DEV

    def body(x_ref, w_ref, out_ref, acc_ref, rs_buf, send_sems, recv_sems):
        my = lax.axis_index("i")
        right = jnp.mod(my + 1, N_DEV)
        left = jnp.mod(my - 1, N_DEV)

        barrier = pltpu.get_barrier_semaphore()
        for nbr in [left, right]:
            pl.semaphore_signal(
                barrier, inc=1,
                device_id=(nbr,), device_id_type=pl.DeviceIdType.MESH,
            )
        pl.semaphore_wait(barrier, 2)

        for c in range(N_DEV):
            sl = pl.ds(c * m_chunk, m_chunk)
            acc_ref[sl, :] = jnp.dot(
                x_ref[sl, :].astype(jnp.bfloat16),
                w_ref[:, :].astype(jnp.bfloat16),
                preferred_element_type=jnp.float32,
            ).astype(jnp.bfloat16)

        for s in range(N_DEV - 1):
            cs = jnp.mod(my - s, N_DEV)
            cr = jnp.mod(my - s - 1, N_DEV)
            rdma = pltpu.make_async_remote_copy(
                src_ref=acc_ref.at[pl.ds(cs * m_chunk, m_chunk), :],
                dst_ref=rs_buf.at[s],
                send_sem=send_sems.at[s],
                recv_sem=recv_sems.at[s],
                device_id=(right,),
                device_id_type=pl.DeviceIdType.MESH,
            )
            rdma.start()
            rdma.wait()
            sl = pl.ds(cr * m_chunk, m_chunk)
            acc_ref[sl, :] = acc_ref[sl, :] + rs_buf[s]

        for t in range(N_DEV - 1):
            g = jnp.mod(my + 1 - t, N_DEV)
            sl_g = pl.ds(g * m_chunk, m_chunk)
            rdma = pltpu.make_async_remote_copy(
                src_ref=acc_ref.at[sl_g, :],
                dst_ref=acc_ref.at[sl_g, :],
                send_sem=send_sems.at[N_DEV - 1 + t],
                recv_sem=recv_sems.at[N_DEV - 1 + t],
                device_id=(right,),
                device_id_type=pl.DeviceIdType.MESH,
            )
            rdma.start()
            rdma.wait()

        c0 = 0.7978845608028654
        for c in range(2 * N_DEV):
            sl = pl.ds(c * (m // (2 * N_DEV)), m // (2 * N_DEV))
            y = acc_ref[sl, :].astype(jnp.float32)
            out_ref[sl, :] = 0.5 * y * (1.0 + jnp.tanh(c0 * (y + 0.044715 * y * y * y)))

    return pl.pallas_call(
        body,
        out_shape=jax.ShapeDtypeStruct((m, n), jnp.float32),
        in_specs=[
            pl.BlockSpec(memory_space=pltpu.VMEM),
            pl.BlockSpec(memory_space=pltpu.VMEM),
        ],
        out_specs=pl.BlockSpec(memory_space=pltpu.VMEM),
        scratch_shapes=[
            pltpu.VMEM((m, n), jnp.bfloat16),
            pltpu.VMEM((N_DEV - 1, m_chunk, n), jnp.bfloat16),
            pltpu.SemaphoreType.DMA((2 * (N_DEV - 1),)),
            pltpu.SemaphoreType.DMA((2 * (N_DEV - 1),)),
        ],
        compiler_params=pltpu.CompilerParams(collective_id=0),
    )(x, w_mat)


# baseline (device time: 341424 ns/iter reference)
import jax
import jax.numpy as jnp
from jax import lax
from jax.experimental import pallas as pl
from jax.experimental.pallas import tpu as pltpu

N_DEV = 4


def kernel(x, w_mat):
    m, k = x.shape
    _, n = w_mat.shape
    m_chunk = m // N_DEV

    def body(x_ref, w_ref, out_ref, rs_buf, send_sems, recv_sems):
        acc_ref = out_ref
        my = lax.axis_index("i")
        right = jnp.mod(my + 1, N_DEV)
        left = jnp.mod(my - 1, N_DEV)

        barrier = pltpu.get_barrier_semaphore()
        for nbr in [left, right]:
            pl.semaphore_signal(
                barrier, inc=1,
                device_id=(nbr,), device_id_type=pl.DeviceIdType.MESH,
            )
        pl.semaphore_wait(barrier, 2)

        for c in range(N_DEV):
            sl = pl.ds(c * m_chunk, m_chunk)
            acc_ref[sl, :] = jnp.dot(
                x_ref[sl, :].astype(jnp.bfloat16),
                w_ref[:, :].astype(jnp.bfloat16),
                preferred_element_type=jnp.float32,
            ).astype(jnp.bfloat16)

        for s in range(N_DEV - 1):
            cs = jnp.mod(my - s, N_DEV)
            cr = jnp.mod(my - s - 1, N_DEV)
            rdma = pltpu.make_async_remote_copy(
                src_ref=acc_ref.at[pl.ds(cs * m_chunk, m_chunk), :],
                dst_ref=rs_buf.at[s],
                send_sem=send_sems.at[s],
                recv_sem=recv_sems.at[s],
                device_id=(right,),
                device_id_type=pl.DeviceIdType.MESH,
            )
            rdma.start()
            rdma.wait()
            sl = pl.ds(cr * m_chunk, m_chunk)
            acc_ref[sl, :] = acc_ref[sl, :] + rs_buf[s]

        for t in range(N_DEV - 1):
            g = jnp.mod(my + 1 - t, N_DEV)
            sl_g = pl.ds(g * m_chunk, m_chunk)
            rdma = pltpu.make_async_remote_copy(
                src_ref=acc_ref.at[sl_g, :],
                dst_ref=acc_ref.at[sl_g, :],
                send_sem=send_sems.at[N_DEV - 1 + t],
                recv_sem=recv_sems.at[N_DEV - 1 + t],
                device_id=(right,),
                device_id_type=pl.DeviceIdType.MESH,
            )
            rdma.start()
            rdma.wait()

        c0 = 0.7978845608028654
        for c in range(2 * N_DEV):
            sl = pl.ds(c * (m // (2 * N_DEV)), m // (2 * N_DEV))
            y = acc_ref[sl, :].astype(jnp.float32)
            g = 0.5 * y * (1.0 + jnp.tanh(c0 * (y + 0.044715 * y * y * y)))
            out_ref[sl, :] = g.astype(jnp.bfloat16)

    return pl.pallas_call(
        body,
        out_shape=jax.ShapeDtypeStruct((m, n), jnp.bfloat16),
        in_specs=[
            pl.BlockSpec(memory_space=pltpu.VMEM),
            pl.BlockSpec(memory_space=pltpu.VMEM),
        ],
        out_specs=pl.BlockSpec(memory_space=pltpu.VMEM),
        scratch_shapes=[
            pltpu.VMEM((N_DEV - 1, m_chunk, n), jnp.bfloat16),
            pltpu.SemaphoreType.DMA((2 * (N_DEV - 1),)),
            pltpu.SemaphoreType.DMA((2 * (N_DEV - 1),)),
        ],
        compiler_params=pltpu.CompilerParams(
            collective_id=0,
            vmem_limit_bytes=60 * 1024 * 1024,
        ),
    )(x, w_mat)


# device time: 193464 ns/iter; 1.7648x vs baseline; 1.7648x over previous
import jax
import jax.numpy as jnp
from jax import lax
from jax.experimental import pallas as pl
from jax.experimental.pallas import tpu as pltpu

N_DEV = 4


def kernel(x, w_mat):
    m, k = x.shape
    _, n = w_mat.shape
    mc = m // N_DEV
    nh = n // 2

    def body(x_ref, w_ref, out_ref, buf_a, buf_b,
             send_a, recv_a, send_b, recv_b):
        my = lax.axis_index("i")
        right = jnp.mod(my + 1, N_DEV)
        left = jnp.mod(my - 1, N_DEV)

        barrier = pltpu.get_barrier_semaphore()
        for nbr in [left, right]:
            pl.semaphore_signal(
                barrier, inc=1,
                device_id=(nbr,), device_id_type=pl.DeviceIdType.MESH,
            )
        pl.semaphore_wait(barrier, 2)

        def compute_chunk(c):
            sl = pl.ds(c * mc, mc)
            out_ref[sl, :] = jnp.dot(
                x_ref[sl, :].astype(jnp.bfloat16),
                w_ref[:, :].astype(jnp.bfloat16),
                preferred_element_type=jnp.float32,
            ).astype(jnp.bfloat16)

        def rs_rdma(s, chunk_a, chunk_b):
            ra = pltpu.make_async_remote_copy(
                src_ref=out_ref.at[pl.ds(chunk_a * mc, mc), pl.ds(0, nh)],
                dst_ref=buf_a.at[s],
                send_sem=send_a.at[s],
                recv_sem=recv_a.at[s],
                device_id=(right,),
                device_id_type=pl.DeviceIdType.MESH,
            )
            rb = pltpu.make_async_remote_copy(
                src_ref=out_ref.at[pl.ds(chunk_b * mc, mc), pl.ds(nh, nh)],
                dst_ref=buf_b.at[s],
                send_sem=send_b.at[s],
                recv_sem=recv_b.at[s],
                device_id=(left,),
                device_id_type=pl.DeviceIdType.MESH,
            )
            return ra, rb

        compute_chunk(my)
        ra, rb = rs_rdma(0, my, my)
        ra.start()
        rb.start()
        compute_chunk(jnp.mod(my + 1, N_DEV))
        compute_chunk(jnp.mod(my - 1, N_DEV))
        compute_chunk(jnp.mod(my + 2, N_DEV))

        for s in range(N_DEV - 1):
            ra.wait()
            rb.wait()
            ca = jnp.mod(my - s - 1, N_DEV)
            cb = jnp.mod(my + s + 1, N_DEV)
            sa = pl.ds(ca * mc, mc)
            sb = pl.ds(cb * mc, mc)
            out_ref[sa, pl.ds(0, nh)] = out_ref[sa, pl.ds(0, nh)] + buf_a[s]
            out_ref[sb, pl.ds(nh, nh)] = out_ref[sb, pl.ds(nh, nh)] + buf_b[s]
            if s < N_DEV - 2:
                ra, rb = rs_rdma(s + 1, ca, cb)
                ra.start()
                rb.start()

        for t in range(N_DEV - 1):
            ga = jnp.mod(my + 1 - t, N_DEV)
            gb = jnp.mod(my - 1 + t, N_DEV)
            ra = pltpu.make_async_remote_copy(
                src_ref=out_ref.at[pl.ds(ga * mc, mc), pl.ds(0, nh)],
                dst_ref=out_ref.at[pl.ds(ga * mc, mc), pl.ds(0, nh)],
                send_sem=send_a.at[N_DEV - 1 + t],
                recv_sem=recv_a.at[N_DEV - 1 + t],
                device_id=(right,),
                device_id_type=pl.DeviceIdType.MESH,
            )
            rb = pltpu.make_async_remote_copy(
                src_ref=out_ref.at[pl.ds(gb * mc, mc), pl.ds(nh, nh)],
                dst_ref=out_ref.at[pl.ds(gb * mc, mc), pl.ds(nh, nh)],
                send_sem=send_b.at[N_DEV - 1 + t],
                recv_sem=recv_b.at[N_DEV - 1 + t],
                device_id=(left,),
                device_id_type=pl.DeviceIdType.MESH,
            )
            ra.start()
            rb.start()
            ra.wait()
            rb.wait()

        c0 = 0.7978845608028654
        for c in range(2 * N_DEV):
            sl = pl.ds(c * (m // (2 * N_DEV)), m // (2 * N_DEV))
            y = out_ref[sl, :].astype(jnp.float32)
            g = 0.5 * y * (1.0 + jnp.tanh(c0 * (y + 0.044715 * y * y * y)))
            out_ref[sl, :] = g.astype(jnp.bfloat16)

    n_sem = 2 * (N_DEV - 1)
    return pl.pallas_call(
        body,
        out_shape=jax.ShapeDtypeStruct((m, n), jnp.bfloat16),
        in_specs=[
            pl.BlockSpec(memory_space=pltpu.VMEM),
            pl.BlockSpec(memory_space=pltpu.VMEM),
        ],
        out_specs=pl.BlockSpec(memory_space=pltpu.VMEM),
        scratch_shapes=[
            pltpu.VMEM((N_DEV - 1, mc, nh), jnp.bfloat16),
            pltpu.VMEM((N_DEV - 1, mc, nh), jnp.bfloat16),
            pltpu.SemaphoreType.DMA((n_sem,)),
            pltpu.SemaphoreType.DMA((n_sem,)),
            pltpu.SemaphoreType.DMA((n_sem,)),
            pltpu.SemaphoreType.DMA((n_sem,)),
        ],
        compiler_params=pltpu.CompilerParams(
            collective_id=0,
            vmem_limit_bytes=60 * 1024 * 1024,
        ),
    )(x, w_mat)
